# Initial kernel scaffold; baseline (speedup 1.0000x reference)
import jax
import jax.numpy as jnp
from jax.experimental import pallas as pl


def kernel(x, edge_index, batch, W1, b1, W2, b2, W3, b3):
    raise NotImplementedError("write your pallas kernel here")


# speedup vs baseline: 158.9897x; 158.9897x over previous
import functools

import jax
import jax.numpy as jnp
from jax import lax
from jax.experimental import pallas as pl
from jax.experimental.pallas import tpu as pltpu
from jax.experimental.pallas import tpu_sc as plsc

_N = 50000
_NP = 50176
_E = 3200000
_CH = 128
_K = 8
_G = 128
_NC, _NS = 2, 16
_NW = _NC * _NS
_GRP = 98
_ECHUNK = _NW * _GRP * _K
_EP = _ECHUNK * _CH
_RPT = _NP // _NS
_NB = 49
_BLK = 1024
_NR = _NP // 128

_F32 = jnp.float32


def _mesh():
    return plsc.VectorSubcoreMesh(
        core_axis_name="c", subcore_axis_name="s",
        num_cores=_NC, num_subcores=_NS)


def _sc_params():
    return pltpu.CompilerParams(use_tc_tiling_on_sc=False)


def _zero_rows(ref, nrows):
    def body(i, _):
        ref[i] = jnp.zeros((16,), _F32)
        return 0
    lax.fori_loop(0, nrows, body, 0)


def _zero_flat(ref, n):
    def body(i, _):
        ref[pl.ds(i * 16, 16)] = jnp.zeros((16,), _F32)
        return 0
    lax.fori_loop(0, n // 16, body, 0)


def _sc_hist(dst2d):
    @functools.partial(
        pl.kernel,
        out_type=jax.ShapeDtypeStruct((_NC * _NP,), _F32),
        mesh=_mesh(),
        compiler_params=_sc_params(),
        scratch_types=[
            pltpu.VMEM((2, _K, _CH), jnp.int32),
            pltpu.VMEM((_CH,), _F32),
            pltpu.VMEM((_RPT,), _F32),
            pltpu.VMEM_SHARED((_NP,), _F32),
            pltpu.SemaphoreType.DMA,
            pltpu.SemaphoreType.DMA,
            pltpu.SemaphoreType.DMA,
        ],
    )
    def body(dst_hbm, out_hbm, di_v, ones_v, stage_v, acc_sh, s_i0, s_i1, s_s):
        c = lax.axis_index("c")
        s = lax.axis_index("s")
        base = (c * _NS + s) * (_GRP * _K)
        sl = pl.ds(s * _RPT, _RPT)
        _zero_flat(stage_v, _RPT)
        for i in range(_CH // 16):
            ones_v[pl.ds(i * 16, 16)] = jnp.ones((16,), _F32)
        pltpu.sync_copy(stage_v, acc_sh.at[sl])
        plsc.subcore_barrier()

        sem_i = (s_i0, s_i1)

        def fire_idx(g, b):
            pltpu.async_copy(
                dst_hbm.at[pl.ds(base + g * _K, _K), :], di_v.at[b], sem_i[b])

        def wait_idx(b):
            pltpu.make_async_copy(
                dst_hbm.at[pl.ds(0, _K), :], di_v.at[b], sem_i[b]).wait()

        def fire_sca(b):
            for j in range(_K):
                pltpu.async_copy(ones_v, acc_sh.at[di_v.at[b, j]], s_s,
                                 add=True)

        def wait_sca(b):
            for j in range(_K):
                pltpu.make_async_copy(
                    ones_v, acc_sh.at[di_v.at[b, j]], s_s).wait()

        fire_idx(0, 0)
        fire_idx(1, 1)
        wait_idx(0)
        fire_sca(0)

        def pair(it, _):
            g = 2 * it + 1
            wait_sca(0)
            fire_idx(g + 1, 0)
            wait_idx(1)
            fire_sca(1)
            wait_sca(1)
            fire_idx(g + 2, 1)
            wait_idx(0)
            fire_sca(0)
            return 0
        lax.fori_loop(0, (_GRP - 2) // 2, pair, 0)

        wait_sca(0)
        wait_idx(1)
        fire_sca(1)
        wait_sca(1)

        plsc.subcore_barrier()
        pltpu.sync_copy(acc_sh.at[sl], stage_v)
        pltpu.sync_copy(stage_v, out_hbm.at[pl.ds(c * _NP + s * _RPT, _RPT)])

    return body(dst2d)


def _sc_scalar(src2d, dst2d, u):
    @functools.partial(
        pl.kernel,
        out_type=jax.ShapeDtypeStruct((_NC * _NP,), _F32),
        mesh=_mesh(),
        compiler_params=_sc_params(),
        scratch_types=[
            pltpu.VMEM((2, _K, _CH), jnp.int32),
            pltpu.VMEM((2, _K, _CH), jnp.int32),
            pltpu.VMEM((2, _K, _CH), _F32),
            pltpu.VMEM((_RPT,), _F32),
            pltpu.VMEM_SHARED((_NP,), _F32),
            pltpu.VMEM_SHARED((_NP,), _F32),
            pltpu.SemaphoreType.DMA,
            pltpu.SemaphoreType.DMA,
            pltpu.SemaphoreType.DMA,
            pltpu.SemaphoreType.DMA,
        ],
    )
    def body(src_hbm, dst_hbm, u_hbm, out_hbm,
             si_v, di_v, vals_v, stage_v, utab_sh, acc_sh,
             s_i0, s_i1, s_g, s_s):
        c = lax.axis_index("c")
        s = lax.axis_index("s")
        base = (c * _NS + s) * (_GRP * _K)
        sl = pl.ds(s * _RPT, _RPT)
        _zero_flat(stage_v, _RPT)
        pltpu.sync_copy(stage_v, acc_sh.at[sl])
        pltpu.sync_copy(u_hbm.at[sl], stage_v)
        pltpu.sync_copy(stage_v, utab_sh.at[sl])
        plsc.subcore_barrier()

        sem_i = (s_i0, s_i1)

        def fire_idx(g, b):
            cb = pl.ds(base + g * _K, _K)
            pltpu.async_copy(src_hbm.at[cb, :], si_v.at[b], sem_i[b])
            pltpu.async_copy(dst_hbm.at[cb, :], di_v.at[b], sem_i[b])

        def wait_idx(b):
            cb = pl.ds(0, _K)
            pltpu.make_async_copy(src_hbm.at[cb, :], si_v.at[b],
                                  sem_i[b]).wait()
            pltpu.make_async_copy(dst_hbm.at[cb, :], di_v.at[b],
                                  sem_i[b]).wait()

        def fire_gat(b):
            for j in range(_K):
                pltpu.async_copy(
                    utab_sh.at[si_v.at[b, j]], vals_v.at[b, j], s_g)

        def wait_gat(b):
            for j in range(_K):
                pltpu.make_async_copy(
                    utab_sh.at[si_v.at[b, j]], vals_v.at[b, j], s_g).wait()

        def fire_sca(b):
            for j in range(_K):
                pltpu.async_copy(
                    vals_v.at[b, j], acc_sh.at[di_v.at[b, j]], s_s, add=True)

        def wait_sca(b):
            for j in range(_K):
                pltpu.make_async_copy(
                    vals_v.at[b, j], acc_sh.at[di_v.at[b, j]], s_s).wait()

        def run(b, g):
            wait_idx(b)
            fire_gat(b)
            wait_gat(b)
            fire_sca(b)

        fire_idx(0, 0)
        fire_idx(1, 1)
        run(0, 0)

        def pair(it, _):
            g = 2 * it + 1
            wait_sca(0)
            fire_idx(g + 1, 0)
            run(1, g)
            wait_sca(1)
            fire_idx(g + 2, 1)
            run(0, g + 1)
            return 0
        lax.fori_loop(0, (_GRP - 2) // 2, pair, 0)

        wait_sca(0)
        run(1, _GRP - 1)
        wait_sca(1)

        plsc.subcore_barrier()
        pltpu.sync_copy(acc_sh.at[sl], stage_v)
        pltpu.sync_copy(stage_v, out_hbm.at[pl.ds(c * _NP + s * _RPT, _RPT)])

    return body(src2d, dst2d, u)


def _sc_vec(src2d, dst2d, q):
    @functools.partial(
        pl.kernel,
        out_type=jax.ShapeDtypeStruct((_NC * _NP, 16), _F32),
        mesh=_mesh(),
        compiler_params=_sc_params(),
        scratch_types=[
            pltpu.VMEM((2, _K, _CH), jnp.int32),
            pltpu.VMEM((2, _K, _CH), jnp.int32),
            pltpu.VMEM((2, _K * _CH, 16), _F32),
            pltpu.VMEM((_RPT // 4, 16), _F32),
            pltpu.VMEM_SHARED((_NP, 16), _F32),
            pltpu.SemaphoreType.DMA,
            pltpu.SemaphoreType.DMA,
            pltpu.SemaphoreType.DMA,
            pltpu.SemaphoreType.DMA,
        ],
    )
    def body(src_hbm, dst_hbm, q_hbm, out_hbm,
             si_v, di_v, rows_v, stage_v, acc_sh, s_i0, s_i1, s_g, s_s):
        c = lax.axis_index("c")
        s = lax.axis_index("s")
        base = (c * _NS + s) * (_GRP * _K)
        qtr = _RPT // 4
        _zero_rows(stage_v, qtr)
        for r in range(4):
            pltpu.sync_copy(
                stage_v, acc_sh.at[pl.ds(s * _RPT + r * qtr, qtr)])
        plsc.subcore_barrier()

        sem_i = (s_i0, s_i1)

        def fire_idx(g, b):
            cb = pl.ds(base + g * _K, _K)
            pltpu.async_copy(src_hbm.at[cb, :], si_v.at[b], sem_i[b])
            pltpu.async_copy(dst_hbm.at[cb, :], di_v.at[b], sem_i[b])

        def wait_idx(b):
            cb = pl.ds(0, _K)
            pltpu.make_async_copy(src_hbm.at[cb, :], si_v.at[b],
                                  sem_i[b]).wait()
            pltpu.make_async_copy(dst_hbm.at[cb, :], di_v.at[b],
                                  sem_i[b]).wait()

        def fire_gat(b):
            for j in range(_K):
                pltpu.async_copy(
                    q_hbm.at[si_v.at[b, j]],
                    rows_v.at[b, pl.ds(j * _CH, _CH), :], s_g)

        def wait_gat(b):
            for j in range(_K):
                pltpu.make_async_copy(
                    q_hbm.at[si_v.at[b, j]],
                    rows_v.at[b, pl.ds(j * _CH, _CH), :], s_g).wait()

        def fire_sca(b):
            for j in range(_K):
                pltpu.async_copy(
                    rows_v.at[b, pl.ds(j * _CH, _CH), :],
                    acc_sh.at[di_v.at[b, j]], s_s, add=True)

        def wait_sca(b):
            for j in range(_K):
                pltpu.make_async_copy(
                    rows_v.at[b, pl.ds(j * _CH, _CH), :],
                    acc_sh.at[di_v.at[b, j]], s_s).wait()

        def run(b, g):
            wait_idx(b)
            fire_gat(b)
            wait_gat(b)
            fire_sca(b)

        fire_idx(0, 0)
        fire_idx(1, 1)
        run(0, 0)

        def pair(it, _):
            g = 2 * it + 1
            wait_sca(0)
            fire_idx(g + 1, 0)
            run(1, g)
            wait_sca(1)
            fire_idx(g + 2, 1)
            run(0, g + 1)
            return 0
        lax.fori_loop(0, (_GRP - 2) // 2, pair, 0)

        wait_sca(0)
        run(1, _GRP - 1)
        wait_sca(1)

        plsc.subcore_barrier()
        for r in range(4):
            pltpu.sync_copy(
                acc_sh.at[pl.ds(s * _RPT + r * qtr, qtr)], stage_v)
            pltpu.sync_copy(
                stage_v,
                out_hbm.at[pl.ds(c * _NP + s * _RPT + r * qtr, qtr), :])

    return body(src2d, dst2d, q)


def _tc_prep1(cnt0, cnt1, xp):
    def body(c0_ref, c1_ref, x_ref, dis_ref, u_ref):
        deg = c0_ref[...] + c1_ref[...] + 1.0
        dis = lax.rsqrt(deg)
        dis_ref[...] = dis
        u_ref[...] = dis * x_ref[...]

    return pl.pallas_call(
        body,
        out_shape=[jax.ShapeDtypeStruct((_NR, 128), _F32),
                   jax.ShapeDtypeStruct((_NR, 128), _F32)],
    )(cnt0, cnt1, xp)


def _tc_prep2(s1a, s1b, dis, xp, W1, b1, W2):
    def body(s1a_ref, s1b_ref, dis_ref, x_ref, w1_ref, b1_ref, w2_ref, q_ref):
        dis = dis_ref[...]
        p = dis * (s1a_ref[...] + s1b_ref[...]) + dis * dis * x_ref[...]
        h1 = jnp.maximum(p * w1_ref[...] + b1_ref[...], 0.0)
        t = jnp.dot(h1, w2_ref[...], preferred_element_type=_F32)
        q_ref[...] = dis * t

    grid = (_NB,)
    return pl.pallas_call(
        body,
        grid=grid,
        in_specs=[
            pl.BlockSpec((_BLK, 1), lambda i: (i, 0)),
            pl.BlockSpec((_BLK, 1), lambda i: (i, 0)),
            pl.BlockSpec((_BLK, 1), lambda i: (i, 0)),
            pl.BlockSpec((_BLK, 1), lambda i: (i, 0)),
            pl.BlockSpec((1, 64), lambda i: (0, 0)),
            pl.BlockSpec((1, 64), lambda i: (0, 0)),
            pl.BlockSpec((64, 16), lambda i: (0, 0)),
        ],
        out_specs=pl.BlockSpec((_BLK, 16), lambda i: (i, 0)),
        out_shape=jax.ShapeDtypeStruct((_NP, 16), _F32),
    )(s1a, s1b, dis, xp, W1, b1, W2)


def _tc_final(s2a, s2b, q, dis, batch2d, b2, W3, b3):
    def body(s2a_ref, s2b_ref, q_ref, dis_ref, b_ref, b2_ref, w3_ref, b3_ref,
             out_ref, acc_ref):
        i = pl.program_id(0)

        @pl.when(i == 0)
        def _():
            acc_ref[...] = jnp.zeros((_G, 2), _F32)

        dis = dis_ref[...]
        o2 = dis * (s2a_ref[...] + s2b_ref[...] + q_ref[...]) + b2_ref[...]
        h2 = jnp.maximum(o2, 0.0)
        h3 = jnp.dot(h2, w3_ref[...], preferred_element_type=_F32) \
            + b3_ref[...]
        gids = lax.broadcasted_iota(jnp.int32, (_BLK, _G), 1)
        oh = (b_ref[...] == gids).astype(_F32)
        hcat = jnp.concatenate(
            [h3, jnp.ones((_BLK, 1), _F32)], axis=1)
        acc_ref[...] += lax.dot_general(
            oh, hcat, (((0,), (0,)), ((), ())),
            preferred_element_type=_F32)

        @pl.when(i == _NB - 1)
        def _():
            a = acc_ref[...]
            out_ref[...] = a[:, 0:1] / jnp.maximum(a[:, 1:2], 1.0)

    grid = (_NB,)
    return pl.pallas_call(
        body,
        grid=grid,
        in_specs=[
            pl.BlockSpec((_BLK, 16), lambda i: (i, 0)),
            pl.BlockSpec((_BLK, 16), lambda i: (i, 0)),
            pl.BlockSpec((_BLK, 16), lambda i: (i, 0)),
            pl.BlockSpec((_BLK, 1), lambda i: (i, 0)),
            pl.BlockSpec((_BLK, 1), lambda i: (i, 0)),
            pl.BlockSpec((1, 16), lambda i: (0, 0)),
            pl.BlockSpec((16, 1), lambda i: (0, 0)),
            pl.BlockSpec((1, 1), lambda i: (0, 0)),
        ],
        out_specs=pl.BlockSpec((_G, 1), lambda i: (0, 0)),
        out_shape=jax.ShapeDtypeStruct((_G, 1), _F32),
        scratch_shapes=[pltpu.VMEM((_G, 2), _F32)],
    )(s2a, s2b, q, dis, batch2d, b2, W3, b3)


def kernel(x, edge_index, batch, W1, b1, W2, b2, W3, b3):
    pad = _NP - _N
    xp = jnp.pad(x[:, 0], (0, pad)).reshape(_NR, 128)
    batch2d = jnp.pad(batch, (0, pad), constant_values=-1)

    epad = _EP - _E
    padidx = _N + (jnp.arange(epad, dtype=jnp.int32) % pad)
    src2d = jnp.concatenate([edge_index[0], padidx]).reshape(_ECHUNK, _CH)
    dst2d = jnp.concatenate([edge_index[1], padidx]).reshape(_ECHUNK, _CH)

    cntp = _sc_hist(dst2d).reshape(_NC, _NR, 128)
    dis, u = _tc_prep1(cntp[0], cntp[1], xp)
    dis_c = dis.reshape(_NP, 1)
    x_c = xp.reshape(_NP, 1)

    s1p = _sc_scalar(src2d, dst2d, u.reshape(_NP)).reshape(_NC, _NP, 1)
    q = _tc_prep2(s1p[0], s1p[1], dis_c, x_c,
                  W1, b1.reshape(1, 64), W2)

    s2p = _sc_vec(src2d, dst2d, q).reshape(_NC, _NP, 16)
    out = _tc_final(s2p[0], s2p[1], q, dis_c,
                    batch2d.reshape(_NP, 1),
                    b2.reshape(1, 16), W3, b3.reshape(1, 1))
    return out


# speedup vs baseline: 204.7273x; 1.2877x over previous
import functools

import jax
import jax.numpy as jnp
from jax import lax
from jax.experimental import pallas as pl
from jax.experimental.pallas import tpu as pltpu
from jax.experimental.pallas import tpu_sc as plsc

_N = 50000
_NP = 50176
_E = 3200000
_CH = 128
_NCH = _E // _CH
_K = 8
_G = 128
_NC, _NS = 2, 16
_NW = _NC * _NS
_GRP = 98
_WCH = _GRP * _K
_MAING = (_NCH - 31 * _WCH) // _K
_PADCH = _NW * _WCH - _NCH
_RPT = _NP // _NS
_NB = 49
_BLK = 1024
_NR = _NP // 128

_F32 = jnp.float32


def _mesh():
    return plsc.VectorSubcoreMesh(
        core_axis_name="c", subcore_axis_name="s",
        num_cores=_NC, num_subcores=_NS)


def _sc_params():
    return pltpu.CompilerParams(use_tc_tiling_on_sc=False)


def _zero_rows(ref, nrows):
    def body(i, _):
        ref[i] = jnp.zeros((16,), _F32)
        return 0
    lax.fori_loop(0, nrows, body, 0)


def _zero_flat(ref, n):
    def body(i, _):
        ref[pl.ds(i * 16, 16)] = jnp.zeros((16,), _F32)
        return 0
    lax.fori_loop(0, n // 16, body, 0)


def _edge_loader(er_hbm, pad_hbm, row, base, is31, si_v, di_v, sem_i):
    def fire_idx(g, b):
        pad = jnp.logical_and(is31, g >= _MAING)

        @pl.when(jnp.logical_not(pad))
        def _():
            cb = pl.ds(base + g * _K, _K)
            pltpu.async_copy(er_hbm.at[0, cb, :], si_v.at[b], sem_i[b])
            pltpu.async_copy(er_hbm.at[1, cb, :], di_v.at[b], sem_i[b])

        @pl.when(pad)
        def _():
            pb = pl.ds((g - _MAING) * _K, _K)
            pltpu.async_copy(pad_hbm.at[pb, :], si_v.at[b], sem_i[b])
            pltpu.async_copy(pad_hbm.at[pb, :], di_v.at[b], sem_i[b])

    def wait_idx(b):
        cb = pl.ds(0, _K)
        pltpu.make_async_copy(er_hbm.at[0, cb, :], si_v.at[b],
                              sem_i[b]).wait()
        pltpu.make_async_copy(er_hbm.at[1, cb, :], di_v.at[b],
                              sem_i[b]).wait()
    del row
    return fire_idx, wait_idx


def _sc_hist(er, padch):
    @functools.partial(
        pl.kernel,
        out_type=jax.ShapeDtypeStruct((_NC * _NP,), _F32),
        mesh=_mesh(),
        compiler_params=_sc_params(),
        scratch_types=[
            pltpu.VMEM((2, _K, _CH), jnp.int32),
            pltpu.VMEM((_CH,), _F32),
            pltpu.VMEM((_RPT,), _F32),
            pltpu.VMEM_SHARED((_NP,), _F32),
            pltpu.SemaphoreType.DMA,
            pltpu.SemaphoreType.DMA,
            pltpu.SemaphoreType.DMA,
        ],
    )
    def body(er_hbm, pad_hbm, out_hbm, di_v, ones_v, stage_v, acc_sh,
             s_i0, s_i1, s_s):
        c = lax.axis_index("c")
        s = lax.axis_index("s")
        w = c * _NS + s
        base = w * _WCH
        is31 = w == _NW - 1
        sl = pl.ds(s * _RPT, _RPT)
        _zero_flat(stage_v, _RPT)
        for i in range(_CH // 16):
            ones_v[pl.ds(i * 16, 16)] = jnp.ones((16,), _F32)
        pltpu.sync_copy(stage_v, acc_sh.at[sl])
        plsc.subcore_barrier()

        sem_i = (s_i0, s_i1)

        def fire_idx(g, b):
            pad = jnp.logical_and(is31, g >= _MAING)

            @pl.when(jnp.logical_not(pad))
            def _():
                pltpu.async_copy(er_hbm.at[1, pl.ds(base + g * _K, _K), :],
                                 di_v.at[b], sem_i[b])

            @pl.when(pad)
            def _():
                pltpu.async_copy(pad_hbm.at[pl.ds((g - _MAING) * _K, _K), :],
                                 di_v.at[b], sem_i[b])

        def wait_idx(b):
            pltpu.make_async_copy(er_hbm.at[1, pl.ds(0, _K), :], di_v.at[b],
                                  sem_i[b]).wait()

        def fire_sca(b):
            for j in range(_K):
                pltpu.async_copy(ones_v, acc_sh.at[di_v.at[b, j]], s_s,
                                 add=True)

        def wait_sca(b):
            for j in range(_K):
                pltpu.make_async_copy(
                    ones_v, acc_sh.at[di_v.at[b, j]], s_s).wait()

        fire_idx(0, 0)
        fire_idx(1, 1)
        wait_idx(0)
        fire_sca(0)

        def pair(it, _):
            g = 2 * it + 1
            wait_sca(0)
            fire_idx(g + 1, 0)
            wait_idx(1)
            fire_sca(1)
            wait_sca(1)
            fire_idx(g + 2, 1)
            wait_idx(0)
            fire_sca(0)
            return 0
        lax.fori_loop(0, (_GRP - 2) // 2, pair, 0)

        wait_sca(0)
        wait_idx(1)
        fire_sca(1)
        wait_sca(1)

        plsc.subcore_barrier()
        pltpu.sync_copy(acc_sh.at[sl], stage_v)
        pltpu.sync_copy(stage_v, out_hbm.at[pl.ds(c * _NP + s * _RPT, _RPT)])

    return body(er, padch)


def _sc_scalar(er, padch, u):
    @functools.partial(
        pl.kernel,
        out_type=jax.ShapeDtypeStruct((_NC * _NP,), _F32),
        mesh=_mesh(),
        compiler_params=_sc_params(),
        scratch_types=[
            pltpu.VMEM((2, _K, _CH), jnp.int32),
            pltpu.VMEM((2, _K, _CH), jnp.int32),
            pltpu.VMEM((2, _K, _CH), _F32),
            pltpu.VMEM((_RPT,), _F32),
            pltpu.VMEM_SHARED((_NP,), _F32),
            pltpu.VMEM_SHARED((_NP,), _F32),
            pltpu.SemaphoreType.DMA,
            pltpu.SemaphoreType.DMA,
            pltpu.SemaphoreType.DMA,
            pltpu.SemaphoreType.DMA,
            pltpu.SemaphoreType.DMA,
        ],
    )
    def body(er_hbm, pad_hbm, u_hbm, out_hbm,
             si_v, di_v, vals_v, stage_v, utab_sh, acc_sh,
             s_i0, s_i1, s_g0, s_g1, s_s):
        c = lax.axis_index("c")
        s = lax.axis_index("s")
        w = c * _NS + s
        base = w * _WCH
        is31 = w == _NW - 1
        sl = pl.ds(s * _RPT, _RPT)
        _zero_flat(stage_v, _RPT)
        pltpu.sync_copy(stage_v, acc_sh.at[sl])
        pltpu.sync_copy(u_hbm.at[sl], stage_v)
        pltpu.sync_copy(stage_v, utab_sh.at[sl])
        plsc.subcore_barrier()

        sem_i = (s_i0, s_i1)
        sem_g = (s_g0, s_g1)
        fire_idx, wait_idx = _edge_loader(
            er_hbm, pad_hbm, 0, base, is31, si_v, di_v, sem_i)

        def fire_gat(b):
            for j in range(_K):
                pltpu.async_copy(
                    utab_sh.at[si_v.at[b, j]], vals_v.at[b, j], sem_g[b])

        def wait_gat(b):
            for j in range(_K):
                pltpu.make_async_copy(
                    utab_sh.at[si_v.at[b, j]], vals_v.at[b, j],
                    sem_g[b]).wait()

        def fire_sca(b):
            for j in range(_K):
                pltpu.async_copy(
                    vals_v.at[b, j], acc_sh.at[di_v.at[b, j]], s_s, add=True)

        def wait_sca(b):
            for j in range(_K):
                pltpu.make_async_copy(
                    vals_v.at[b, j], acc_sh.at[di_v.at[b, j]], s_s).wait()

        fire_idx(0, 0)
        wait_idx(0)
        fire_gat(0)
        fire_idx(1, 1)
        wait_gat(0)
        fire_sca(0)
        wait_idx(1)
        fire_gat(1)

        def steady(g, b):
            nb = 1 - b
            wait_sca(nb)
            fire_idx(g + 1, nb)
            wait_gat(b)
            fire_sca(b)
            wait_idx(nb)
            fire_gat(nb)

        def pair(it, _):
            g = 2 * it + 1
            steady(g, 1)
            steady(g + 1, 0)
            return 0
        lax.fori_loop(0, (_GRP - 2) // 2, pair, 0)

        wait_sca(0)
        wait_gat(1)
        fire_sca(1)
        wait_sca(1)

        plsc.subcore_barrier()
        pltpu.sync_copy(acc_sh.at[sl], stage_v)
        pltpu.sync_copy(stage_v, out_hbm.at[pl.ds(c * _NP + s * _RPT, _RPT)])

    return body(er, padch, u)


def _sc_vec(er, padch, q):
    @functools.partial(
        pl.kernel,
        out_type=jax.ShapeDtypeStruct((_NC * _NP, 16), _F32),
        mesh=_mesh(),
        compiler_params=_sc_params(),
        scratch_types=[
            pltpu.VMEM((2, _K, _CH), jnp.int32),
            pltpu.VMEM((2, _K, _CH), jnp.int32),
            pltpu.VMEM((2, _K * _CH, 16), _F32),
            pltpu.VMEM((_RPT // 4, 16), _F32),
            pltpu.VMEM_SHARED((_NP, 16), _F32),
            pltpu.SemaphoreType.DMA,
            pltpu.SemaphoreType.DMA,
            pltpu.SemaphoreType.DMA,
            pltpu.SemaphoreType.DMA,
            pltpu.SemaphoreType.DMA,
        ],
    )
    def body(er_hbm, pad_hbm, q_hbm, out_hbm,
             si_v, di_v, rows_v, stage_v, acc_sh,
             s_i0, s_i1, s_g0, s_g1, s_s):
        c = lax.axis_index("c")
        s = lax.axis_index("s")
        w = c * _NS + s
        base = w * _WCH
        is31 = w == _NW - 1
        qtr = _RPT // 4
        _zero_rows(stage_v, qtr)
        for r in range(4):
            pltpu.sync_copy(
                stage_v, acc_sh.at[pl.ds(s * _RPT + r * qtr, qtr)])
        plsc.subcore_barrier()

        sem_i = (s_i0, s_i1)
        sem_g = (s_g0, s_g1)
        fire_idx, wait_idx = _edge_loader(
            er_hbm, pad_hbm, 0, base, is31, si_v, di_v, sem_i)

        def fire_gat(b):
            for j in range(_K):
                pltpu.async_copy(
                    q_hbm.at[si_v.at[b, j]],
                    rows_v.at[b, pl.ds(j * _CH, _CH), :], sem_g[b])

        def wait_gat(b):
            for j in range(_K):
                pltpu.make_async_copy(
                    q_hbm.at[si_v.at[b, j]],
                    rows_v.at[b, pl.ds(j * _CH, _CH), :], sem_g[b]).wait()

        def fire_sca(b):
            for j in range(_K):
                pltpu.async_copy(
                    rows_v.at[b, pl.ds(j * _CH, _CH), :],
                    acc_sh.at[di_v.at[b, j]], s_s, add=True)

        def wait_sca(b):
            for j in range(_K):
                pltpu.make_async_copy(
                    rows_v.at[b, pl.ds(j * _CH, _CH), :],
                    acc_sh.at[di_v.at[b, j]], s_s).wait()

        fire_idx(0, 0)
        wait_idx(0)
        fire_gat(0)
        fire_idx(1, 1)
        wait_gat(0)
        fire_sca(0)
        wait_idx(1)
        fire_gat(1)

        def steady(g, b):
            nb = 1 - b
            wait_sca(nb)
            fire_idx(g + 1, nb)
            wait_gat(b)
            fire_sca(b)
            wait_idx(nb)
            fire_gat(nb)

        def pair(it, _):
            g = 2 * it + 1
            steady(g, 1)
            steady(g + 1, 0)
            return 0
        lax.fori_loop(0, (_GRP - 2) // 2, pair, 0)

        wait_sca(0)
        wait_gat(1)
        fire_sca(1)
        wait_sca(1)

        plsc.subcore_barrier()
        for r in range(4):
            pltpu.sync_copy(
                acc_sh.at[pl.ds(s * _RPT + r * qtr, qtr)], stage_v)
            pltpu.sync_copy(
                stage_v,
                out_hbm.at[pl.ds(c * _NP + s * _RPT + r * qtr, qtr), :])

    return body(er, padch, q)


def _tc_prep1(cnt, xp):
    def body(c_ref, x_ref, dis_ref, u_ref):
        deg = c_ref[0:_NR] + c_ref[_NR:2 * _NR] + 1.0
        dis = lax.rsqrt(deg)
        dis_ref[...] = dis
        u_ref[...] = dis * x_ref[...]

    return pl.pallas_call(
        body,
        out_shape=[jax.ShapeDtypeStruct((_NR, 128), _F32),
                   jax.ShapeDtypeStruct((_NR, 128), _F32)],
    )(cnt, xp)


def _tc_prep2(s1p, dis, xp, W1, b1, W2):
    def body(s1a_ref, s1b_ref, dis_ref, x_ref, w1_ref, b1_ref, w2_ref, q_ref):
        dis = dis_ref[...]
        p = dis * (s1a_ref[...] + s1b_ref[...]) + dis * dis * x_ref[...]
        h1 = jnp.maximum(p * w1_ref[...] + b1_ref[...], 0.0)
        t = jnp.dot(h1, w2_ref[...], preferred_element_type=_F32)
        q_ref[...] = dis * t

    grid = (_NB,)
    return pl.pallas_call(
        body,
        grid=grid,
        in_specs=[
            pl.BlockSpec((_BLK, 1), lambda i: (i, 0)),
            pl.BlockSpec((_BLK, 1), lambda i: (_NB + i, 0)),
            pl.BlockSpec((_BLK, 1), lambda i: (i, 0)),
            pl.BlockSpec((_BLK, 1), lambda i: (i, 0)),
            pl.BlockSpec((1, 64), lambda i: (0, 0)),
            pl.BlockSpec((1, 64), lambda i: (0, 0)),
            pl.BlockSpec((64, 16), lambda i: (0, 0)),
        ],
        out_specs=pl.BlockSpec((_BLK, 16), lambda i: (i, 0)),
        out_shape=jax.ShapeDtypeStruct((_NP, 16), _F32),
    )(s1p, s1p, dis, xp, W1, b1, W2)


def _tc_final(s2p, q, dis, batch2d, b2, W3, b3):
    def body(s2a_ref, s2b_ref, q_ref, dis_ref, b_ref, b2_ref, w3_ref, b3_ref,
             out_ref, acc_ref):
        i = pl.program_id(0)

        @pl.when(i == 0)
        def _():
            acc_ref[...] = jnp.zeros((_G, 2), _F32)

        dis = dis_ref[...]
        o2 = dis * (s2a_ref[...] + s2b_ref[...] + q_ref[...]) + b2_ref[...]
        h2 = jnp.maximum(o2, 0.0)
        h3 = jnp.dot(h2, w3_ref[...], preferred_element_type=_F32) \
            + b3_ref[...]
        gids = lax.broadcasted_iota(jnp.int32, (_BLK, _G), 1)
        oh = (b_ref[...] == gids).astype(_F32)
        hcat = jnp.concatenate(
            [h3, jnp.ones((_BLK, 1), _F32)], axis=1)
        acc_ref[...] += lax.dot_general(
            oh, hcat, (((0,), (0,)), ((), ())),
            preferred_element_type=_F32)

        @pl.when(i == _NB - 1)
        def _():
            a = acc_ref[...]
            out_ref[...] = a[:, 0:1] / jnp.maximum(a[:, 1:2], 1.0)

    grid = (_NB,)
    return pl.pallas_call(
        body,
        grid=grid,
        in_specs=[
            pl.BlockSpec((_BLK, 16), lambda i: (i, 0)),
            pl.BlockSpec((_BLK, 16), lambda i: (_NB + i, 0)),
            pl.BlockSpec((_BLK, 16), lambda i: (i, 0)),
            pl.BlockSpec((_BLK, 1), lambda i: (i, 0)),
            pl.BlockSpec((_BLK, 1), lambda i: (i, 0)),
            pl.BlockSpec((1, 16), lambda i: (0, 0)),
            pl.BlockSpec((16, 1), lambda i: (0, 0)),
            pl.BlockSpec((1, 1), lambda i: (0, 0)),
        ],
        out_specs=pl.BlockSpec((_G, 1), lambda i: (0, 0)),
        out_shape=jax.ShapeDtypeStruct((_G, 1), _F32),
        scratch_shapes=[pltpu.VMEM((_G, 2), _F32)],
    )(s2p, s2p, q, dis, batch2d, b2, W3, b3)


def kernel(x, edge_index, batch, W1, b1, W2, b2, W3, b3):
    pad = _NP - _N
    xp = jnp.pad(x[:, 0], (0, pad)).reshape(_NR, 128)
    batch2d = jnp.pad(batch, (0, pad), constant_values=-1)

    er = edge_index.reshape(2, _NCH, _CH)
    padch = (_N + (jnp.arange(_PADCH * _CH, dtype=jnp.int32) % pad)
             ).reshape(_PADCH, _CH)

    cntp = _sc_hist(er, padch).reshape(2 * _NR, 128)
    dis, u = _tc_prep1(cntp, xp)
    dis_c = dis.reshape(_NP, 1)
    x_c = xp.reshape(_NP, 1)

    s1p = _sc_scalar(er, padch, u.reshape(_NP)).reshape(_NC * _NP, 1)
    q = _tc_prep2(s1p, dis_c, x_c, W1, b1.reshape(1, 64), W2)

    s2p = _sc_vec(er, padch, q)
    out = _tc_final(s2p, q, dis_c, batch2d.reshape(_NP, 1),
                    b2.reshape(1, 16), W3, b3.reshape(1, 1))
    return out
